# Initial kernel scaffold; baseline (speedup 1.0000x reference)
import jax
import jax.numpy as jnp
from jax.experimental import pallas as pl


def kernel(users, ui_edge_index, ui_edge_vals, ua_edge_index, ua_edge_vals, g2_rows, g2_cols, g2_vals, g4_rows, g4_cols, g4_vals, vlogger_list, user_emb, video_emb, vlogger_emb, q):
    raise NotImplementedError("write your pallas kernel here")


# speedup vs baseline: 1.0370x; 1.0370x over previous
import functools

import jax
import jax.numpy as jnp
from jax.experimental import pallas as pl
from jax.experimental.pallas import tpu as pltpu


def _spmm(rows, cols, vals, x, n_out):
    return jnp.zeros((n_out, x.shape[1]), x.dtype).at[rows].add(vals[:, None] * x[cols])


_BU = 512
_BV = 1024

_DN = (((1,), (0,)), ((), ()))


def _rating_body(u0_ref, u1_ref, avt_ref, nvt_ref, vl0t_ref, vl1t_ref, qt_ref,
                 out_ref):
    u0 = u0_ref[...]
    u1 = u1_ref[...]
    avt = avt_ref[...]
    nvt = nvt_ref[...]
    vl0t = vl0t_ref[...]
    vl1t = vl1t_ref[...]
    vft = (avt + nvt) * 0.5
    wft = (vl0t + vl1t) * 0.5
    ct = jax.lax.dot_general(qt_ref[...], vft, _DN,
                             preferred_element_type=jnp.float32)
    w = jax.nn.sigmoid(jnp.sum(ct * wft, axis=0, keepdims=True))
    ui = jax.nn.sigmoid(
        jax.lax.dot_general(u0, avt, _DN, preferred_element_type=jnp.float32)
        + jax.lax.dot_general(u1, nvt, _DN, preferred_element_type=jnp.float32))
    ua = jax.nn.sigmoid(
        jax.lax.dot_general(u0, vl0t, _DN, preferred_element_type=jnp.float32)
        + jax.lax.dot_general(u1, vl1t, _DN, preferred_element_type=jnp.float32))
    out_ref[...] = w * ui + (1.0 - w) * ua


def _rating(u0, u1, avt, nvt, vl0t, vl1t, qt):
    B, D = u0.shape
    V = avt.shape[1]
    grid = (pl.cdiv(B, _BU), pl.cdiv(V, _BV))
    per_u = pl.BlockSpec((_BU, D), lambda i, j: (i, 0))
    per_v = pl.BlockSpec((D, _BV), lambda i, j: (0, j))
    return pl.pallas_call(
        _rating_body,
        grid=grid,
        in_specs=[per_u, per_u, per_v, per_v, per_v, per_v,
                  pl.BlockSpec((D, D), lambda i, j: (0, 0))],
        out_specs=pl.BlockSpec((_BU, _BV), lambda i, j: (i, j)),
        out_shape=jax.ShapeDtypeStruct((B, V), jnp.float32),
    )(u0, u1, avt, nvt, vl0t, vl1t, qt)


def _normalize(x):
    n = jnp.sqrt(jnp.sum(x * x, axis=1, keepdims=True))
    return x / jnp.maximum(n, 1e-12)


def kernel(users, ui_edge_index, ui_edge_vals, ua_edge_index, ua_edge_vals,
           g2_rows, g2_cols, g2_vals, g4_rows, g4_cols, g4_vals,
           vlogger_list, user_emb, video_emb, vlogger_emb, q):
    U, D = user_emb.shape
    V = video_emb.shape[0]
    A = vlogger_emb.shape[0]

    feats_ui = jnp.concatenate([user_emb, video_emb], axis=0)
    p_ui = _spmm(ui_edge_index[0], ui_edge_index[1], ui_edge_vals, feats_ui, U + V)
    light_ui = (feats_ui + 2.0 * p_ui) * (1.0 / 3.0)
    atom_users, atom_videos = light_ui[:U], light_ui[U:]

    atom_vloggers = _normalize(_spmm(g2_rows, g2_cols, g2_vals, atom_videos, A))

    feats_ua = jnp.concatenate([user_emb, vlogger_emb], axis=0)
    p_ua = _spmm(ua_edge_index[0], ua_edge_index[1], ua_edge_vals, feats_ua, U + A)
    light_ua = (feats_ua + 2.0 * p_ua) * (1.0 / 3.0)
    non_users, non_vloggers = light_ua[:U], light_ua[U:]

    non_videos = _normalize(_spmm(g4_rows, g4_cols, g4_vals, non_users, V))

    u0 = atom_users[users]
    u1 = non_users[users]
    vl0 = atom_vloggers[vlogger_list]
    vl1 = non_vloggers[vlogger_list]

    return _rating(u0, u1, atom_videos.T, non_videos.T, vl0.T, vl1.T, q.T)


# speedup vs baseline: 2.0026x; 1.9312x over previous
import functools

import jax
import jax.numpy as jnp
from jax import lax
from jax.experimental import pallas as pl
from jax.experimental.pallas import tpu as pltpu
from jax.experimental.pallas import tpu_sc as plsc

_NS = 16
_NC = 2
_ZW = 16384


def _make_sc_spmm(E_pad, N_in, N_out_pad, F, n_rowsplit):
    n_chunks = 64 // F
    R = N_out_pad // n_rowsplit
    n_jobs = n_chunks * n_rowsplit
    jobs_per_core = n_jobs // _NC
    EW = E_pad // _NS
    n_blk = EW // 2048
    rps = R // _NS

    mesh = plsc.VectorSubcoreMesh(core_axis_name="c", subcore_axis_name="s")

    @functools.partial(
        pl.kernel,
        out_type=jax.ShapeDtypeStruct((n_chunks, N_out_pad, F), jnp.float32),
        mesh=mesh,
        scratch_types=[
            pltpu.VMEM((_NS, 128), jnp.int32),
            pltpu.VMEM((_NS, 128), jnp.int32),
            pltpu.VMEM((_NS, 128), jnp.float32),
            pltpu.VMEM((128, F), jnp.float32),
            pltpu.VMEM_SHARED((R, F), jnp.float32),
            pltpu.SemaphoreType.DMA,
        ],
        compiler_params=pltpu.CompilerParams(use_tc_tiling_on_sc=False),
    )
    def spmm(rows_hbm, cols_hbm, vals_hbm, xs_hbm, zeros_hbm, out_hbm,
             rbuf, cbuf, vbuf, fbuf, acc, sem):
        c = lax.axis_index("c")
        s = lax.axis_index("s")

        lane = lax.iota(jnp.int32, 16)
        zero16 = (lane * 0).astype(jnp.float32)

        def scale_block(j, lo):
            for m in range(8):
                if n_rowsplit > 1:
                    rv = rbuf[j, pl.ds(m * 16, 16)]
                    inr = jnp.logical_and(rv >= lo, rv < lo + R)
                    rbuf[j, pl.ds(m * 16, 16)] = jnp.clip(rv - lo, 0, R - 1)
                    vv = jnp.where(inr, vbuf[j, pl.ds(m * 16, 16)], zero16)
                else:
                    vv = vbuf[j, pl.ds(m * 16, 16)]
                for t in range(16):
                    e = m * 16 + t
                    sp = vv.at[lane * 0 + t].get(mode="promise_in_bounds")
                    for h in range(F // 16):
                        fbuf[e, pl.ds(h * 16, 16)] = (
                            fbuf[e, pl.ds(h * 16, 16)] * sp)

        for p in range(jobs_per_core):
            job = c * jobs_per_core + p
            if n_rowsplit > 1:
                chunk = job >> 1
                lo = (job & 1) * R
            else:
                chunk = job
                lo = 0
            z0 = s * rps
            pltpu.sync_copy(zeros_hbm, acc.at[pl.ds(z0, rps)])
            plsc.subcore_barrier()

            xv = xs_hbm.at[chunk]

            def blk_body(b, _):
                rb = s * (EW // 128) + b * 16
                pltpu.sync_copy(rows_hbm.at[pl.ds(rb, _NS)], rbuf)
                pltpu.sync_copy(cols_hbm.at[pl.ds(rb, _NS)], cbuf)
                pltpu.sync_copy(vals_hbm.at[pl.ds(rb, _NS)], vbuf)

                def sub_body(j, _):
                    pltpu.async_copy(xv.at[cbuf.at[j]], fbuf, sem).wait()
                    scale_block(j, lo)
                    pltpu.sync_copy(fbuf, acc.at[rbuf.at[j]], add=True)
                    return 0
                lax.fori_loop(0, _NS, sub_body, 0)
                return 0
            lax.fori_loop(0, n_blk, blk_body, 0)
            plsc.subcore_barrier()

            pltpu.sync_copy(acc.at[pl.ds(z0, rps)],
                            out_hbm.at[chunk].at[pl.ds(lo + z0, rps)])
            plsc.subcore_barrier()

    return spmm


def _pad_edges(rows, cols, vals, E_pad):
    E = rows.shape[0]
    pad = E_pad - E
    rows = jnp.concatenate([rows, jnp.zeros((pad,), rows.dtype)])
    cols = jnp.concatenate([cols, jnp.zeros((pad,), cols.dtype)])
    vals = jnp.concatenate([vals, jnp.zeros((pad,), vals.dtype)])
    return (rows.reshape(-1, 128), cols.reshape(-1, 128),
            vals.reshape(-1, 128))


def _sc_spmm(rows, cols, vals, x, n_out, F, n_rowsplit):
    E = rows.shape[0]
    E_pad = -(-E // (_NS * 2048)) * (_NS * 2048)
    align = _NS * 8 * n_rowsplit
    n_out_pad = -(-n_out // align) * align
    n_chunks = 64 // F
    N_in = x.shape[0]
    r2, c2, v2 = _pad_edges(rows, cols, vals, E_pad)
    xs = x.reshape(N_in, n_chunks, F).transpose(1, 0, 2)
    zeros = jnp.zeros((n_out_pad // n_rowsplit // _NS, F), jnp.float32)
    fn = _make_sc_spmm(E_pad, N_in, n_out_pad, F, n_rowsplit)
    out = fn(r2, c2, v2, xs, zeros)
    return out.transpose(1, 0, 2).reshape(n_out_pad, 64)[:n_out]


_BU = 512
_BV = 1024

_DN = (((1,), (0,)), ((), ()))


def _rating_body(u0_ref, u1_ref, avt_ref, nvt_ref, vl0t_ref, vl1t_ref, qt_ref,
                 out_ref):
    u0 = u0_ref[...]
    u1 = u1_ref[...]
    avt = avt_ref[...]
    nvt = nvt_ref[...]
    vl0t = vl0t_ref[...]
    vl1t = vl1t_ref[...]
    vft = (avt + nvt) * 0.5
    wft = (vl0t + vl1t) * 0.5
    ct = jax.lax.dot_general(qt_ref[...], vft, _DN,
                             preferred_element_type=jnp.float32)
    w = jax.nn.sigmoid(jnp.sum(ct * wft, axis=0, keepdims=True))
    ui = jax.nn.sigmoid(
        jax.lax.dot_general(u0, avt, _DN, preferred_element_type=jnp.float32)
        + jax.lax.dot_general(u1, nvt, _DN, preferred_element_type=jnp.float32))
    ua = jax.nn.sigmoid(
        jax.lax.dot_general(u0, vl0t, _DN, preferred_element_type=jnp.float32)
        + jax.lax.dot_general(u1, vl1t, _DN, preferred_element_type=jnp.float32))
    out_ref[...] = w * ui + (1.0 - w) * ua


def _rating(u0, u1, avt, nvt, vl0t, vl1t, qt):
    B, D = u0.shape
    V = avt.shape[1]
    grid = (pl.cdiv(B, _BU), pl.cdiv(V, _BV))
    per_u = pl.BlockSpec((_BU, D), lambda i, j: (i, 0))
    per_v = pl.BlockSpec((D, _BV), lambda i, j: (0, j))
    return pl.pallas_call(
        _rating_body,
        grid=grid,
        in_specs=[per_u, per_u, per_v, per_v, per_v, per_v,
                  pl.BlockSpec((D, D), lambda i, j: (0, 0))],
        out_specs=pl.BlockSpec((_BU, _BV), lambda i, j: (i, j)),
        out_shape=jax.ShapeDtypeStruct((B, V), jnp.float32),
    )(u0, u1, avt, nvt, vl0t, vl1t, qt)


def _normalize(x):
    n = jnp.sqrt(jnp.sum(x * x, axis=1, keepdims=True))
    return x / jnp.maximum(n, 1e-12)


def kernel(users, ui_edge_index, ui_edge_vals, ua_edge_index, ua_edge_vals,
           g2_rows, g2_cols, g2_vals, g4_rows, g4_cols, g4_vals,
           vlogger_list, user_emb, video_emb, vlogger_emb, q):
    U, D = user_emb.shape
    V = video_emb.shape[0]
    A = vlogger_emb.shape[0]

    feats_ui = jnp.concatenate([user_emb, video_emb], axis=0)
    p_ui = _sc_spmm(ui_edge_index[0], ui_edge_index[1], ui_edge_vals,
                    feats_ui, U + V, 16, 2)
    light_ui = (feats_ui + 2.0 * p_ui) * (1.0 / 3.0)
    atom_users, atom_videos = light_ui[:U], light_ui[U:]

    atom_vloggers = _normalize(_sc_spmm(g2_rows, g2_cols, g2_vals,
                                        atom_videos, A, 32, 1))

    feats_ua = jnp.concatenate([user_emb, vlogger_emb], axis=0)
    p_ua = _sc_spmm(ua_edge_index[0], ua_edge_index[1], ua_edge_vals,
                    feats_ua, U + A, 16, 1)
    light_ua = (feats_ua + 2.0 * p_ua) * (1.0 / 3.0)
    non_users, non_vloggers = light_ua[:U], light_ua[U:]

    non_videos = _normalize(_sc_spmm(g4_rows, g4_cols, g4_vals,
                                     non_users, V, 32, 1))

    u0 = atom_users[users]
    u1 = non_users[users]
    vl0 = atom_vloggers[vlogger_list]
    vl1 = non_vloggers[vlogger_list]

    return _rating(u0, u1, atom_videos.T, non_videos.T, vl0.T, vl1.T, q.T)


# speedup vs baseline: 2.1635x; 1.0803x over previous
import functools

import jax
import jax.numpy as jnp
from jax import lax
from jax.experimental import pallas as pl
from jax.experimental.pallas import tpu as pltpu
from jax.experimental.pallas import tpu_sc as plsc

_NS = 16
_NC = 2
_ZW = 16384


def _make_sc_spmm(E_pad, N_in, N_out_pad, F, n_rowsplit):
    n_chunks = 64 // F
    R = N_out_pad // n_rowsplit
    n_jobs = n_chunks * n_rowsplit
    jobs_per_core = n_jobs // _NC
    EW = E_pad // _NS
    n_blk = EW // 2048
    rps = R // _NS

    mesh = plsc.VectorSubcoreMesh(core_axis_name="c", subcore_axis_name="s")

    @functools.partial(
        pl.kernel,
        out_type=jax.ShapeDtypeStruct((n_chunks, N_out_pad, F), jnp.float32),
        mesh=mesh,
        scratch_types=[
            pltpu.VMEM((_NS, 128), jnp.int32),
            pltpu.VMEM((_NS, 128), jnp.int32),
            pltpu.VMEM((_NS, 128), jnp.float32),
            pltpu.VMEM((4, 128, F), jnp.float32),
            pltpu.VMEM_SHARED((R, F), jnp.float32),
            pltpu.SemaphoreType.DMA((4,)),
            pltpu.SemaphoreType.DMA((4,)),
        ],
        compiler_params=pltpu.CompilerParams(use_tc_tiling_on_sc=False),
    )
    def spmm(rows_hbm, cols_hbm, vals_hbm, xs_hbm, zeros_hbm, out_hbm,
             rbuf, cbuf, vbuf, fbuf, acc, gsem, ssem):
        c = lax.axis_index("c")
        s = lax.axis_index("s")

        lane = lax.iota(jnp.int32, 16)
        zero16 = (lane * 0).astype(jnp.float32)

        def scale_block(j, g, lo):
            for m in range(8):
                if n_rowsplit > 1:
                    rv = rbuf[j, pl.ds(m * 16, 16)]
                    inr = jnp.logical_and(rv >= lo, rv < lo + R)
                    rbuf[j, pl.ds(m * 16, 16)] = jnp.clip(rv - lo, 0, R - 1)
                    vv = jnp.where(inr, vbuf[j, pl.ds(m * 16, 16)], zero16)
                else:
                    vv = vbuf[j, pl.ds(m * 16, 16)]
                for t in range(16):
                    e = m * 16 + t
                    sp = vv.at[lane * 0 + t].get(mode="promise_in_bounds")
                    for h in range(F // 16):
                        fbuf[g, e, pl.ds(h * 16, 16)] = (
                            fbuf[g, e, pl.ds(h * 16, 16)] * sp)

        for p in range(jobs_per_core):
            job = c * jobs_per_core + p
            if n_rowsplit > 1:
                chunk = job >> 1
                lo = (job & 1) * R
            else:
                chunk = job
                lo = 0
            z0 = s * rps
            pltpu.sync_copy(zeros_hbm, acc.at[pl.ds(z0, rps)])
            plsc.subcore_barrier()

            xv = xs_hbm.at[chunk]

            def blk_body(b, _):
                rb = s * (EW // 128) + b * 16
                pltpu.sync_copy(rows_hbm.at[pl.ds(rb, _NS)], rbuf)
                pltpu.sync_copy(cols_hbm.at[pl.ds(rb, _NS)], cbuf)
                pltpu.sync_copy(vals_hbm.at[pl.ds(rb, _NS)], vbuf)

                def sub_body(i, _):
                    gs = [pltpu.async_copy(xv.at[cbuf.at[4 * i + g]],
                                           fbuf.at[g], gsem.at[g])
                          for g in range(4)]
                    ss = []
                    for g in range(4):
                        j = 4 * i + g
                        gs[g].wait()
                        scale_block(j, g, lo)
                        ss.append(pltpu.async_copy(
                            fbuf.at[g], acc.at[rbuf.at[j]], ssem.at[g],
                            add=True))
                    for d in ss:
                        d.wait()
                    return 0
                lax.fori_loop(0, _NS // 4, sub_body, 0)
                return 0
            lax.fori_loop(0, n_blk, blk_body, 0)
            plsc.subcore_barrier()

            pltpu.sync_copy(acc.at[pl.ds(z0, rps)],
                            out_hbm.at[chunk].at[pl.ds(lo + z0, rps)])
            plsc.subcore_barrier()

    return spmm


def _pad_edges(rows, cols, vals, E_pad):
    E = rows.shape[0]
    pad = E_pad - E
    rows = jnp.concatenate([rows, jnp.zeros((pad,), rows.dtype)])
    cols = jnp.concatenate([cols, jnp.zeros((pad,), cols.dtype)])
    vals = jnp.concatenate([vals, jnp.zeros((pad,), vals.dtype)])
    return (rows.reshape(-1, 128), cols.reshape(-1, 128),
            vals.reshape(-1, 128))


def _sc_spmm(rows, cols, vals, x, n_out, F, n_rowsplit):
    E = rows.shape[0]
    E_pad = -(-E // (_NS * 2048)) * (_NS * 2048)
    align = _NS * 8 * n_rowsplit
    n_out_pad = -(-n_out // align) * align
    n_chunks = 64 // F
    N_in = x.shape[0]
    r2, c2, v2 = _pad_edges(rows, cols, vals, E_pad)
    xs = x.reshape(N_in, n_chunks, F).transpose(1, 0, 2)
    zeros = jnp.zeros((n_out_pad // n_rowsplit // _NS, F), jnp.float32)
    fn = _make_sc_spmm(E_pad, N_in, n_out_pad, F, n_rowsplit)
    out = fn(r2, c2, v2, xs, zeros)
    return out.transpose(1, 0, 2).reshape(n_out_pad, 64)[:n_out]


_BU = 512
_BV = 1024

_DN = (((1,), (0,)), ((), ()))


def _rating_body(u0_ref, u1_ref, avt_ref, nvt_ref, vl0t_ref, vl1t_ref, qt_ref,
                 out_ref):
    u0 = u0_ref[...]
    u1 = u1_ref[...]
    avt = avt_ref[...]
    nvt = nvt_ref[...]
    vl0t = vl0t_ref[...]
    vl1t = vl1t_ref[...]
    vft = (avt + nvt) * 0.5
    wft = (vl0t + vl1t) * 0.5
    ct = jax.lax.dot_general(qt_ref[...], vft, _DN,
                             preferred_element_type=jnp.float32)
    w = jax.nn.sigmoid(jnp.sum(ct * wft, axis=0, keepdims=True))
    ui = jax.nn.sigmoid(
        jax.lax.dot_general(u0, avt, _DN, preferred_element_type=jnp.float32)
        + jax.lax.dot_general(u1, nvt, _DN, preferred_element_type=jnp.float32))
    ua = jax.nn.sigmoid(
        jax.lax.dot_general(u0, vl0t, _DN, preferred_element_type=jnp.float32)
        + jax.lax.dot_general(u1, vl1t, _DN, preferred_element_type=jnp.float32))
    out_ref[...] = w * ui + (1.0 - w) * ua


def _rating(u0, u1, avt, nvt, vl0t, vl1t, qt):
    B, D = u0.shape
    V = avt.shape[1]
    grid = (pl.cdiv(B, _BU), pl.cdiv(V, _BV))
    per_u = pl.BlockSpec((_BU, D), lambda i, j: (i, 0))
    per_v = pl.BlockSpec((D, _BV), lambda i, j: (0, j))
    return pl.pallas_call(
        _rating_body,
        grid=grid,
        in_specs=[per_u, per_u, per_v, per_v, per_v, per_v,
                  pl.BlockSpec((D, D), lambda i, j: (0, 0))],
        out_specs=pl.BlockSpec((_BU, _BV), lambda i, j: (i, j)),
        out_shape=jax.ShapeDtypeStruct((B, V), jnp.float32),
    )(u0, u1, avt, nvt, vl0t, vl1t, qt)


def _normalize(x):
    n = jnp.sqrt(jnp.sum(x * x, axis=1, keepdims=True))
    return x / jnp.maximum(n, 1e-12)


def kernel(users, ui_edge_index, ui_edge_vals, ua_edge_index, ua_edge_vals,
           g2_rows, g2_cols, g2_vals, g4_rows, g4_cols, g4_vals,
           vlogger_list, user_emb, video_emb, vlogger_emb, q):
    U, D = user_emb.shape
    V = video_emb.shape[0]
    A = vlogger_emb.shape[0]

    feats_ui = jnp.concatenate([user_emb, video_emb], axis=0)
    p_ui = _sc_spmm(ui_edge_index[0], ui_edge_index[1], ui_edge_vals,
                    feats_ui, U + V, 16, 2)
    light_ui = (feats_ui + 2.0 * p_ui) * (1.0 / 3.0)
    atom_users, atom_videos = light_ui[:U], light_ui[U:]

    atom_vloggers = _normalize(_sc_spmm(g2_rows, g2_cols, g2_vals,
                                        atom_videos, A, 32, 1))

    feats_ua = jnp.concatenate([user_emb, vlogger_emb], axis=0)
    p_ua = _sc_spmm(ua_edge_index[0], ua_edge_index[1], ua_edge_vals,
                    feats_ua, U + A, 16, 1)
    light_ua = (feats_ua + 2.0 * p_ua) * (1.0 / 3.0)
    non_users, non_vloggers = light_ua[:U], light_ua[U:]

    non_videos = _normalize(_sc_spmm(g4_rows, g4_cols, g4_vals,
                                     non_users, V, 32, 1))

    u0 = atom_users[users]
    u1 = non_users[users]
    vl0 = atom_vloggers[vlogger_list]
    vl1 = non_vloggers[vlogger_list]

    return _rating(u0, u1, atom_videos.T, non_videos.T, vl0.T, vl1.T, q.T)
